# Initial kernel scaffold; baseline (speedup 1.0000x reference)
import functools

import jax
import jax.numpy as jnp
from jax import lax
from jax.experimental import pallas as pl
from jax.experimental.pallas import tpu as pltpu
from jax.experimental.pallas import tpu_sc as plsc

N = 10000
E = 320000
IN = 128
HID = 128
OUT = 64
B = 256

NC = 2
NS = 16
NW = NC * NS
CHUNK = 128
K = -(-E // (NW * CHUNK))
EPAD = NW * CHUNK * K
ROWS_PER_TILE = 320
NPAD = NW * ROWS_PER_TILE
ZROWS = 64
NB = 256
NGRID = NPAD // NB

_mesh = plsc.VectorSubcoreMesh(core_axis_name="c", subcore_axis_name="s",
                               num_cores=NC, num_subcores=NS)


def _sc_agg_body(want_deg, table, src3, dst3, *rest):
    if want_deg:
        (out_sum, out_deg, src_idx, dst_idx, rows, zbuf, sum_sh, sem,
         ones16, zbuf16, deg_sh) = rest
    else:
        out_sum, src_idx, dst_idx, rows, zbuf, sum_sh, sem = rest
    c = lax.axis_index("c")
    s = lax.axis_index("s")
    wid = c * NS + s

    zero16 = jnp.zeros((16,), jnp.float32)

    def zero_zbuf(i, _):
        q = i // (HID // 16)
        r = i % (HID // 16)
        zbuf[q, pl.ds(r * 16, 16)] = zero16
        return 0

    lax.fori_loop(0, ZROWS * (HID // 16), zero_zbuf, 0)

    def zero_sum(j, _):
        pltpu.sync_copy(zbuf, sum_sh.at[pl.ds(s * ROWS_PER_TILE + j * ZROWS, ZROWS)])
        return 0

    lax.fori_loop(0, ROWS_PER_TILE // ZROWS, zero_sum, 0)

    if want_deg:
        one16 = jnp.ones((16,), jnp.float32)

        def fill_small(i, _):
            zbuf16[i] = zero16
            ones16[i % CHUNK] = one16
            return 0

        lax.fori_loop(0, ROWS_PER_TILE, fill_small, 0)
        pltpu.sync_copy(zbuf16, deg_sh.at[pl.ds(s * ROWS_PER_TILE, ROWS_PER_TILE)])

    plsc.subcore_barrier()

    pltpu.sync_copy(src3.at[wid], src_idx)
    pltpu.sync_copy(dst3.at[wid], dst_idx)

    def step(k, _):
        pltpu.async_copy(table.at[src_idx.at[k]], rows, sem).wait()
        pltpu.sync_copy(rows, sum_sh.at[dst_idx.at[k]], add=True)
        if want_deg:
            pltpu.sync_copy(ones16, deg_sh.at[dst_idx.at[k]], add=True)
        return 0

    lax.fori_loop(0, K, step, 0)

    plsc.subcore_barrier()

    off = s * ROWS_PER_TILE
    hoff = c * NPAD + off
    pltpu.sync_copy(sum_sh.at[pl.ds(off, ROWS_PER_TILE)],
                    out_sum.at[pl.ds(hoff, ROWS_PER_TILE)])
    if want_deg:
        pltpu.sync_copy(deg_sh.at[pl.ds(off, ROWS_PER_TILE)],
                        out_deg.at[pl.ds(hoff, ROWS_PER_TILE)])


_sc_agg_deg = pl.kernel(
    functools.partial(_sc_agg_body, True),
    out_type=[jax.ShapeDtypeStruct((NC * NPAD, HID), jnp.float32),
              jax.ShapeDtypeStruct((NC * NPAD, 16), jnp.float32)],
    mesh=_mesh,
    scratch_types=[
        pltpu.VMEM((K, CHUNK), jnp.int32),
        pltpu.VMEM((K, CHUNK), jnp.int32),
        pltpu.VMEM((CHUNK, HID), jnp.float32),
        pltpu.VMEM((ZROWS, HID), jnp.float32),
        pltpu.VMEM_SHARED((NPAD, HID), jnp.float32),
        pltpu.SemaphoreType.DMA,
        pltpu.VMEM((CHUNK, 16), jnp.float32),
        pltpu.VMEM((ROWS_PER_TILE, 16), jnp.float32),
        pltpu.VMEM_SHARED((NPAD, 16), jnp.float32),
    ],
)

_sc_agg = pl.kernel(
    functools.partial(_sc_agg_body, False),
    out_type=[jax.ShapeDtypeStruct((NC * NPAD, HID), jnp.float32)],
    mesh=_mesh,
    scratch_types=[
        pltpu.VMEM((K, CHUNK), jnp.int32),
        pltpu.VMEM((K, CHUNK), jnp.int32),
        pltpu.VMEM((CHUNK, HID), jnp.float32),
        pltpu.VMEM((ZROWS, HID), jnp.float32),
        pltpu.VMEM_SHARED((NPAD, HID), jnp.float32),
        pltpu.SemaphoreType.DMA,
    ],
)

_P = jax.lax.Precision.HIGHEST


def _h1_body(sum_ref, deg_ref, x_ref, wl_ref, wr_ref, b_ref, out_ref):
    ssum = sum_ref[0] + sum_ref[1]
    deg = deg_ref[0, :, 0:1] + deg_ref[1, :, 0:1]
    mean = ssum / jnp.maximum(deg, 1.0)
    h = (jnp.dot(mean, wl_ref[...], preferred_element_type=jnp.float32, precision=_P)
         + jnp.dot(x_ref[...], wr_ref[...], preferred_element_type=jnp.float32, precision=_P)
         + b_ref[...])
    out_ref[...] = jnp.maximum(h, 0.0)


def _final_body(sum_ref, deg_ref, h1_ref, p_ref, batch_ref, root_ref,
                wl_ref, wr_ref, b2_ref, wtop_ref, wbot_ref, blin_ref,
                out_ref, pooled_acc, cnt_acc, roots_acc):
    i = pl.program_id(0)

    @pl.when(i == 0)
    def _():
        pooled_acc[...] = jnp.zeros_like(pooled_acc)
        cnt_acc[...] = jnp.zeros_like(cnt_acc)
        roots_acc[...] = jnp.zeros_like(roots_acc)

    ssum = sum_ref[0] + sum_ref[1]
    deg = deg_ref[0, :, 0:1] + deg_ref[1, :, 0:1]
    mean2 = ssum / jnp.maximum(deg, 1.0)
    h2 = (jnp.dot(mean2, wl_ref[...], preferred_element_type=jnp.float32, precision=_P)
          + jnp.dot(h1_ref[...], wr_ref[...], preferred_element_type=jnp.float32, precision=_P)
          + b2_ref[...])

    iota_b = lax.broadcasted_iota(jnp.int32, (NB, B), 1)
    mask = (batch_ref[...] == iota_b).astype(jnp.float32)
    pw = mask * p_ref[...]
    dg = lambda a, b: lax.dot_general(a, b, (((0,), (0,)), ((), ())),
                                      preferred_element_type=jnp.float32,
                                      precision=_P)
    pooled_acc[...] += dg(pw, h2)
    cnt_acc[...] += dg(mask, jnp.ones((NB, HID), jnp.float32))
    rowid = i * NB + lax.broadcasted_iota(jnp.int32, (NB, B), 0)
    rmask = (rowid == root_ref[...]).astype(jnp.float32)
    roots_acc[...] += dg(rmask, h2)

    @pl.when(i == NGRID - 1)
    def _():
        pooled = pooled_acc[...] / jnp.maximum(cnt_acc[...], 1.0)
        out_ref[...] = (
            jnp.dot(roots_acc[...], wtop_ref[...], preferred_element_type=jnp.float32, precision=_P)
            + jnp.dot(pooled, wbot_ref[...], preferred_element_type=jnp.float32, precision=_P)
            + blin_ref[...])


def kernel(x, edge_index, p, batch, root_ptr,
           W_l1, W_r1, b1, W_l2, W_r2, b2, W_lin, b_lin):
    f32 = jnp.float32
    src = edge_index[0].astype(jnp.int32)
    dst = edge_index[1].astype(jnp.int32)
    src3 = jnp.concatenate([src, jnp.zeros((EPAD - E,), jnp.int32)]).reshape(NW, K, CHUNK)
    dst3 = jnp.concatenate([dst, jnp.full((EPAD - E,), N, jnp.int32)]).reshape(NW, K, CHUNK)
    x_pad = jnp.zeros((NPAD, IN), f32).at[:N].set(x)

    sum1, deg16 = _sc_agg_deg(x_pad, src3, dst3)
    sum1 = sum1.reshape(NC, NPAD, HID)
    deg16 = deg16.reshape(NC, NPAD, 16)

    h1 = pl.pallas_call(
        _h1_body,
        grid=(NGRID,),
        in_specs=[
            pl.BlockSpec((NC, NB, HID), lambda i: (0, i, 0)),
            pl.BlockSpec((NC, NB, 16), lambda i: (0, i, 0)),
            pl.BlockSpec((NB, IN), lambda i: (i, 0)),
            pl.BlockSpec((IN, HID), lambda i: (0, 0)),
            pl.BlockSpec((IN, HID), lambda i: (0, 0)),
            pl.BlockSpec((1, HID), lambda i: (0, 0)),
        ],
        out_specs=pl.BlockSpec((NB, HID), lambda i: (i, 0)),
        out_shape=jax.ShapeDtypeStruct((NPAD, HID), f32),
    )(sum1, deg16, x_pad, W_l1, W_r1, b1.reshape(1, HID))

    (sum2,) = _sc_agg(h1, src3, dst3)
    sum2 = sum2.reshape(NC, NPAD, HID)

    p_pad = jnp.zeros((NPAD, 1), f32).at[:N, 0].set(p)
    batch_pad = jnp.full((NPAD, 1), B, jnp.int32).at[:N, 0].set(batch.astype(jnp.int32))
    root_row = root_ptr.astype(jnp.int32).reshape(1, B)

    out = pl.pallas_call(
        _final_body,
        grid=(NGRID,),
        in_specs=[
            pl.BlockSpec((NC, NB, HID), lambda i: (0, i, 0)),
            pl.BlockSpec((NC, NB, 16), lambda i: (0, i, 0)),
            pl.BlockSpec((NB, HID), lambda i: (i, 0)),
            pl.BlockSpec((NB, 1), lambda i: (i, 0)),
            pl.BlockSpec((NB, 1), lambda i: (i, 0)),
            pl.BlockSpec((1, B), lambda i: (0, 0)),
            pl.BlockSpec((HID, HID), lambda i: (0, 0)),
            pl.BlockSpec((HID, HID), lambda i: (0, 0)),
            pl.BlockSpec((1, HID), lambda i: (0, 0)),
            pl.BlockSpec((HID, OUT), lambda i: (0, 0)),
            pl.BlockSpec((HID, OUT), lambda i: (0, 0)),
            pl.BlockSpec((1, OUT), lambda i: (0, 0)),
        ],
        out_specs=pl.BlockSpec((B, OUT), lambda i: (0, 0)),
        out_shape=jax.ShapeDtypeStruct((B, OUT), f32),
        scratch_shapes=[
            pltpu.VMEM((B, HID), f32),
            pltpu.VMEM((B, HID), f32),
            pltpu.VMEM((B, HID), f32),
        ],
    )(sum2, deg16, h1, p_pad, batch_pad, root_row,
      W_l2, W_r2, b2.reshape(1, HID), W_lin[:HID], W_lin[HID:],
      b_lin.reshape(1, OUT))

    return out


# speedup vs baseline: 3.9703x; 3.9703x over previous
import functools

import jax
import jax.numpy as jnp
from jax import lax
from jax.experimental import pallas as pl
from jax.experimental.pallas import tpu as pltpu
from jax.experimental.pallas import tpu_sc as plsc

N = 10000
E = 320000
IN = 128
HID = 128
OUT = 64
B = 256

NC = 2
NS = 16
NW = NC * NS
CHUNK = 128
K = -(-E // (NW * CHUNK))
EPAD = NW * CHUNK * K
NPAD = 10240
ROWS_PER_TILE = NPAD // NS
ZROWS = 16
NB = 256
NGRID = NPAD // NB

_mesh = plsc.VectorSubcoreMesh(core_axis_name="c", subcore_axis_name="s",
                               num_cores=NC, num_subcores=NS)


def _sc_agg_body(want_deg, table, src3, dst3, *rest):
    if want_deg:
        out_sum, out_deg, src_idx, dst_idx, rows, zbuf, sum_sh, sem = rest
    else:
        out_sum, src_idx, dst_idx, rows, zbuf, sum_sh, sem = rest
    c = lax.axis_index("c")
    s = lax.axis_index("s")
    wid = c * NS + s
    off = s * ROWS_PER_TILE
    hoff = c * NPAD + off
    zero16 = jnp.zeros((16,), jnp.float32)

    def zero_zbuf(i, _):
        zbuf[i // 8, pl.ds((i % 8) * 16, 16)] = zero16
        return 0

    def zero_table(j, _):
        pltpu.sync_copy(zbuf, sum_sh.at[pl.ds(off + j * ZROWS, ZROWS)])
        return 0

    lax.fori_loop(0, ZROWS * (HID // 16), zero_zbuf, 0)
    lax.fori_loop(0, ROWS_PER_TILE // ZROWS, zero_table, 0)
    plsc.subcore_barrier()

    def step(k, _):
        row = wid * K + k
        pltpu.sync_copy(src3.at[row], src_idx)
        pltpu.sync_copy(dst3.at[row], dst_idx)
        pltpu.async_copy(table.at[src_idx], rows, sem).wait()
        pltpu.sync_copy(rows, sum_sh.at[dst_idx], add=True)
        return 0

    lax.fori_loop(0, K, step, 0)
    plsc.subcore_barrier()
    pltpu.sync_copy(sum_sh.at[pl.ds(off, ROWS_PER_TILE)],
                    out_sum.at[pl.ds(hoff, ROWS_PER_TILE)])

    if want_deg:
        plsc.subcore_barrier()
        lax.fori_loop(0, ROWS_PER_TILE // ZROWS, zero_table, 0)

        one16 = jnp.ones((16,), jnp.float32)

        def fill_ones(i, _):
            rows[i // 8, pl.ds((i % 8) * 16, 16)] = one16
            return 0

        lax.fori_loop(0, CHUNK * (HID // 16), fill_ones, 0)
        plsc.subcore_barrier()

        def step_deg(k, _):
            pltpu.sync_copy(dst3.at[wid * K + k], dst_idx)
            pltpu.sync_copy(rows, sum_sh.at[dst_idx], add=True)
            return 0

        lax.fori_loop(0, K, step_deg, 0)
        plsc.subcore_barrier()
        pltpu.sync_copy(sum_sh.at[pl.ds(off, ROWS_PER_TILE)],
                        out_deg.at[pl.ds(hoff, ROWS_PER_TILE)])


_sc_agg_deg = pl.kernel(
    functools.partial(_sc_agg_body, True),
    out_type=[jax.ShapeDtypeStruct((NC * NPAD, HID), jnp.float32),
              jax.ShapeDtypeStruct((NC * NPAD, HID), jnp.float32)],
    mesh=_mesh,
    scratch_types=[
        pltpu.VMEM((CHUNK,), jnp.int32),
        pltpu.VMEM((CHUNK,), jnp.int32),
        pltpu.VMEM((CHUNK, HID), jnp.float32),
        pltpu.VMEM((ZROWS, HID), jnp.float32),
        pltpu.VMEM_SHARED((NPAD, HID), jnp.float32),
        pltpu.SemaphoreType.DMA,
    ],
)

_sc_agg = pl.kernel(
    functools.partial(_sc_agg_body, False),
    out_type=[jax.ShapeDtypeStruct((NC * NPAD, HID), jnp.float32)],
    mesh=_mesh,
    scratch_types=[
        pltpu.VMEM((CHUNK,), jnp.int32),
        pltpu.VMEM((CHUNK,), jnp.int32),
        pltpu.VMEM((CHUNK, HID), jnp.float32),
        pltpu.VMEM((ZROWS, HID), jnp.float32),
        pltpu.VMEM_SHARED((NPAD, HID), jnp.float32),
        pltpu.SemaphoreType.DMA,
    ],
)

_P = jax.lax.Precision.HIGHEST


def _h1_body(sum_ref, deg_ref, x_ref, wl_ref, wr_ref, b_ref, out_ref):
    ssum = sum_ref[0] + sum_ref[1]
    deg = deg_ref[0, :, 0:1] + deg_ref[1, :, 0:1]
    mean = ssum / jnp.maximum(deg, 1.0)
    h = (jnp.dot(mean, wl_ref[...], preferred_element_type=jnp.float32, precision=_P)
         + jnp.dot(x_ref[...], wr_ref[...], preferred_element_type=jnp.float32, precision=_P)
         + b_ref[...])
    out_ref[...] = jnp.maximum(h, 0.0)


def _final_body(sum_ref, deg_ref, h1_ref, p_ref, batch_ref, root_ref,
                wl_ref, wr_ref, b2_ref, wtop_ref, wbot_ref, blin_ref,
                out_ref, pooled_acc, cnt_acc, roots_acc):
    i = pl.program_id(0)

    @pl.when(i == 0)
    def _():
        pooled_acc[...] = jnp.zeros_like(pooled_acc)
        cnt_acc[...] = jnp.zeros_like(cnt_acc)
        roots_acc[...] = jnp.zeros_like(roots_acc)

    ssum = sum_ref[0] + sum_ref[1]
    deg = deg_ref[0, :, 0:1] + deg_ref[1, :, 0:1]
    mean2 = ssum / jnp.maximum(deg, 1.0)
    h2 = (jnp.dot(mean2, wl_ref[...], preferred_element_type=jnp.float32, precision=_P)
          + jnp.dot(h1_ref[...], wr_ref[...], preferred_element_type=jnp.float32, precision=_P)
          + b2_ref[...])

    iota_b = lax.broadcasted_iota(jnp.int32, (NB, B), 1)
    mask = (batch_ref[...] == iota_b).astype(jnp.float32)
    pw = mask * p_ref[...]
    dg = lambda a, b: lax.dot_general(a, b, (((0,), (0,)), ((), ())),
                                      preferred_element_type=jnp.float32,
                                      precision=_P)
    pooled_acc[...] += dg(pw, h2)
    cnt_acc[...] += dg(mask, jnp.ones((NB, HID), jnp.float32))
    rowid = i * NB + lax.broadcasted_iota(jnp.int32, (NB, B), 0)
    rmask = (rowid == root_ref[...]).astype(jnp.float32)
    roots_acc[...] += dg(rmask, h2)

    @pl.when(i == NGRID - 1)
    def _():
        pooled = pooled_acc[...] / jnp.maximum(cnt_acc[...], 1.0)
        out_ref[...] = (
            jnp.dot(roots_acc[...], wtop_ref[...], preferred_element_type=jnp.float32, precision=_P)
            + jnp.dot(pooled, wbot_ref[...], preferred_element_type=jnp.float32, precision=_P)
            + blin_ref[...])


def kernel(x, edge_index, p, batch, root_ptr,
           W_l1, W_r1, b1, W_l2, W_r2, b2, W_lin, b_lin):
    f32 = jnp.float32
    src = edge_index[0].astype(jnp.int32)
    dst = edge_index[1].astype(jnp.int32)
    src3 = jnp.concatenate([src, jnp.zeros((EPAD - E,), jnp.int32)]).reshape(NW * K, CHUNK)
    dst3 = jnp.concatenate([dst, jnp.full((EPAD - E,), N, jnp.int32)]).reshape(NW * K, CHUNK)
    x_pad = jnp.zeros((NPAD, IN), f32).at[:N].set(x)

    sum1, deg = _sc_agg_deg(x_pad, src3, dst3)
    sum1 = sum1.reshape(NC, NPAD, HID)
    deg = deg.reshape(NC, NPAD, HID)

    h1 = pl.pallas_call(
        _h1_body,
        grid=(NGRID,),
        in_specs=[
            pl.BlockSpec((NC, NB, HID), lambda i: (0, i, 0)),
            pl.BlockSpec((NC, NB, HID), lambda i: (0, i, 0)),
            pl.BlockSpec((NB, IN), lambda i: (i, 0)),
            pl.BlockSpec((IN, HID), lambda i: (0, 0)),
            pl.BlockSpec((IN, HID), lambda i: (0, 0)),
            pl.BlockSpec((1, HID), lambda i: (0, 0)),
        ],
        out_specs=pl.BlockSpec((NB, HID), lambda i: (i, 0)),
        out_shape=jax.ShapeDtypeStruct((NPAD, HID), f32),
    )(sum1, deg, x_pad, W_l1, W_r1, b1.reshape(1, HID))

    (sum2,) = _sc_agg(h1, src3, dst3)
    sum2 = sum2.reshape(NC, NPAD, HID)

    p_pad = jnp.zeros((NPAD, 1), f32).at[:N, 0].set(p)
    batch_pad = jnp.full((NPAD, 1), B, jnp.int32).at[:N, 0].set(batch.astype(jnp.int32))
    root_row = root_ptr.astype(jnp.int32).reshape(1, B)

    out = pl.pallas_call(
        _final_body,
        grid=(NGRID,),
        in_specs=[
            pl.BlockSpec((NC, NB, HID), lambda i: (0, i, 0)),
            pl.BlockSpec((NC, NB, HID), lambda i: (0, i, 0)),
            pl.BlockSpec((NB, HID), lambda i: (i, 0)),
            pl.BlockSpec((NB, 1), lambda i: (i, 0)),
            pl.BlockSpec((NB, 1), lambda i: (i, 0)),
            pl.BlockSpec((1, B), lambda i: (0, 0)),
            pl.BlockSpec((HID, HID), lambda i: (0, 0)),
            pl.BlockSpec((HID, HID), lambda i: (0, 0)),
            pl.BlockSpec((1, HID), lambda i: (0, 0)),
            pl.BlockSpec((HID, OUT), lambda i: (0, 0)),
            pl.BlockSpec((HID, OUT), lambda i: (0, 0)),
            pl.BlockSpec((1, OUT), lambda i: (0, 0)),
        ],
        out_specs=pl.BlockSpec((B, OUT), lambda i: (0, 0)),
        out_shape=jax.ShapeDtypeStruct((B, OUT), f32),
        scratch_shapes=[
            pltpu.VMEM((B, HID), f32),
            pltpu.VMEM((B, HID), f32),
            pltpu.VMEM((B, HID), f32),
        ],
    )(sum2, deg, h1, p_pad, batch_pad, root_row,
      W_l2, W_r2, b2.reshape(1, HID), W_lin[:HID], W_lin[HID:],
      b_lin.reshape(1, OUT))

    return out
